# Initial kernel scaffold; baseline (speedup 1.0000x reference)
import jax
import jax.numpy as jnp
from jax.experimental import pallas as pl


def kernel(W1, W2, Wmid0, x, adj):
    raise NotImplementedError("write your pallas kernel here")


# speedup vs baseline: 3.0955x; 3.0955x over previous
import functools

import jax
import jax.numpy as jnp
from jax.experimental import pallas as pl
from jax.experimental.pallas import tpu as pltpu

_VMEM_LIMIT_BYTES = 64 * 1024 * 1024


def _layer_kernel(adj_ref, b_ref, w_ref, out_ref):
    h = jnp.dot(adj_ref[...], b_ref[...], preferred_element_type=jnp.float32)
    h = jnp.maximum(h, 0.0).astype(jnp.bfloat16)
    out_ref[...] = jnp.dot(
        h, w_ref[...], preferred_element_type=jnp.float32
    ).astype(out_ref.dtype)


def _final_kernel(adj_ref, b_ref, out_ref):
    out_ref[...] = jnp.dot(
        adj_ref[...], b_ref[...], preferred_element_type=jnp.float32
    ).astype(out_ref.dtype)


def _fused_layer(adj_b, b, w, *, tm, out_dtype):
    m, k = adj_b.shape
    h = b.shape[1]
    c = w.shape[1]
    return pl.pallas_call(
        _layer_kernel,
        out_shape=jax.ShapeDtypeStruct((m, c), out_dtype),
        grid=(m // tm,),
        in_specs=[
            pl.BlockSpec((tm, k), lambda i: (i, 0)),
            pl.BlockSpec((k, h), lambda i: (0, 0)),
            pl.BlockSpec((h, c), lambda i: (0, 0)),
        ],
        out_specs=pl.BlockSpec((tm, c), lambda i: (i, 0)),
        compiler_params=pltpu.CompilerParams(
            dimension_semantics=("parallel",),
            vmem_limit_bytes=_VMEM_LIMIT_BYTES,
        ),
    )(adj_b, b, w)


def _final_matmul(adj_b, b, *, tm, out_dtype):
    m, k = adj_b.shape
    c = b.shape[1]
    return pl.pallas_call(
        _final_kernel,
        out_shape=jax.ShapeDtypeStruct((m, c), out_dtype),
        grid=(m // tm,),
        in_specs=[
            pl.BlockSpec((tm, k), lambda i: (i, 0)),
            pl.BlockSpec((k, c), lambda i: (0, 0)),
        ],
        out_specs=pl.BlockSpec((tm, c), lambda i: (i, 0)),
        compiler_params=pltpu.CompilerParams(
            dimension_semantics=("parallel",),
            vmem_limit_bytes=_VMEM_LIMIT_BYTES,
        ),
    )(adj_b, b)


def kernel(W1, W2, Wmid0, x, adj):
    del x
    n = adj.shape[0]
    assert n % 256 == 0, adj.shape
    tm = 256

    adj_b = adj.astype(jnp.bfloat16)
    w1_b = W1.astype(jnp.bfloat16)
    wm_b = Wmid0.astype(jnp.bfloat16)
    w2_b = W2.astype(jnp.bfloat16)

    pre1 = _fused_layer(adj_b, w1_b, wm_b, tm=tm, out_dtype=jnp.bfloat16)
    pre2 = _fused_layer(adj_b, pre1, w2_b, tm=tm, out_dtype=jnp.bfloat16)
    return _final_matmul(adj_b, pre2, tm=tm, out_dtype=jnp.float32)


# speedup vs baseline: 3.5388x; 1.1432x over previous
import functools

import jax
import jax.numpy as jnp
from jax.experimental import pallas as pl
from jax.experimental.pallas import tpu as pltpu

_VMEM_LIMIT_BYTES = 64 * 1024 * 1024


def _layer1_kernel(adj_ref, b_ref, w_ref, adjb_ref, out_ref):
    adj_b = adj_ref[...].astype(jnp.bfloat16)
    adjb_ref[...] = adj_b
    h = jnp.dot(adj_b, b_ref[...], preferred_element_type=jnp.float32)
    h = jnp.maximum(h, 0.0).astype(jnp.bfloat16)
    out_ref[...] = jnp.dot(
        h, w_ref[...], preferred_element_type=jnp.float32
    ).astype(out_ref.dtype)


def _layer_kernel(adj_ref, b_ref, w_ref, out_ref):
    h = jnp.dot(adj_ref[...], b_ref[...], preferred_element_type=jnp.float32)
    h = jnp.maximum(h, 0.0).astype(jnp.bfloat16)
    out_ref[...] = jnp.dot(
        h, w_ref[...], preferred_element_type=jnp.float32
    ).astype(out_ref.dtype)


def _final_kernel(adj_ref, b_ref, out_ref):
    out_ref[...] = jnp.dot(
        adj_ref[...], b_ref[...], preferred_element_type=jnp.float32
    ).astype(out_ref.dtype)


def _layer1(adj_f32, b, w, *, tm):
    m, k = adj_f32.shape
    h = b.shape[1]
    c = w.shape[1]
    return pl.pallas_call(
        _layer1_kernel,
        out_shape=(
            jax.ShapeDtypeStruct((m, k), jnp.bfloat16),
            jax.ShapeDtypeStruct((m, c), jnp.bfloat16),
        ),
        grid=(m // tm,),
        in_specs=[
            pl.BlockSpec((tm, k), lambda i: (i, 0)),
            pl.BlockSpec((k, h), lambda i: (0, 0)),
            pl.BlockSpec((h, c), lambda i: (0, 0)),
        ],
        out_specs=(
            pl.BlockSpec((tm, k), lambda i: (i, 0)),
            pl.BlockSpec((tm, c), lambda i: (i, 0)),
        ),
        compiler_params=pltpu.CompilerParams(
            dimension_semantics=("parallel",),
            vmem_limit_bytes=_VMEM_LIMIT_BYTES,
        ),
    )(adj_f32, b, w)


def _fused_layer(adj_b, b, w, *, tm, out_dtype):
    m, k = adj_b.shape
    h = b.shape[1]
    c = w.shape[1]
    return pl.pallas_call(
        _layer_kernel,
        out_shape=jax.ShapeDtypeStruct((m, c), out_dtype),
        grid=(m // tm,),
        in_specs=[
            pl.BlockSpec((tm, k), lambda i: (i, 0)),
            pl.BlockSpec((k, h), lambda i: (0, 0)),
            pl.BlockSpec((h, c), lambda i: (0, 0)),
        ],
        out_specs=pl.BlockSpec((tm, c), lambda i: (i, 0)),
        compiler_params=pltpu.CompilerParams(
            dimension_semantics=("parallel",),
            vmem_limit_bytes=_VMEM_LIMIT_BYTES,
        ),
    )(adj_b, b, w)


def _final_matmul(adj_b, b, *, tm, out_dtype):
    m, k = adj_b.shape
    c = b.shape[1]
    return pl.pallas_call(
        _final_kernel,
        out_shape=jax.ShapeDtypeStruct((m, c), out_dtype),
        grid=(m // tm,),
        in_specs=[
            pl.BlockSpec((tm, k), lambda i: (i, 0)),
            pl.BlockSpec((k, c), lambda i: (0, 0)),
        ],
        out_specs=pl.BlockSpec((tm, c), lambda i: (i, 0)),
        compiler_params=pltpu.CompilerParams(
            dimension_semantics=("parallel",),
            vmem_limit_bytes=_VMEM_LIMIT_BYTES,
        ),
    )(adj_b, b)


def kernel(W1, W2, Wmid0, x, adj):
    del x
    n = adj.shape[0]
    assert n % 256 == 0, adj.shape
    tm = 256

    w1_b = W1.astype(jnp.bfloat16)
    wm_b = Wmid0.astype(jnp.bfloat16)
    w2_b = W2.astype(jnp.bfloat16)

    adj_b, pre1 = _layer1(adj, w1_b, wm_b, tm=tm)
    pre2 = _fused_layer(adj_b, pre1, w2_b, tm=tm, out_dtype=jnp.bfloat16)
    return _final_matmul(adj_b, pre2, tm=tm, out_dtype=jnp.float32)


# speedup vs baseline: 4.1161x; 1.1631x over previous
import functools

import jax
import jax.numpy as jnp
from jax.experimental import pallas as pl
from jax.experimental.pallas import tpu as pltpu

_VMEM_LIMIT_BYTES = 64 * 1024 * 1024


def _layer1_kernel(adj_ref, b_ref, w_ref, adjb_ref, out_ref):
    adj_b = adj_ref[...].astype(jnp.bfloat16)
    adjb_ref[...] = adj_b
    h = jnp.dot(adj_b, b_ref[...], preferred_element_type=jnp.float32)
    h = jnp.maximum(h, 0.0).astype(jnp.bfloat16)
    out_ref[...] = jnp.dot(
        h, w_ref[...], preferred_element_type=jnp.float32
    ).astype(out_ref.dtype)


def _layer_kernel(adj_ref, b_ref, w_ref, out_ref):
    h = jnp.dot(adj_ref[...], b_ref[...], preferred_element_type=jnp.float32)
    h = jnp.maximum(h, 0.0).astype(jnp.bfloat16)
    out_ref[...] = jnp.dot(
        h, w_ref[...], preferred_element_type=jnp.float32
    ).astype(out_ref.dtype)


def _final_kernel(adj_ref, b_ref, out_ref):
    out_ref[...] = jnp.dot(
        adj_ref[...], b_ref[...], preferred_element_type=jnp.float32
    ).astype(out_ref.dtype)


def _layer1(adj_f32, b, w, *, tm):
    m, k = adj_f32.shape
    h = b.shape[1]
    c = w.shape[1]
    return pl.pallas_call(
        _layer1_kernel,
        out_shape=(
            jax.ShapeDtypeStruct((m, k), jnp.bfloat16),
            jax.ShapeDtypeStruct((m, c), jnp.bfloat16),
        ),
        grid=(m // tm,),
        in_specs=[
            pl.BlockSpec((tm, k), lambda i: (i, 0)),
            pl.BlockSpec((k, h), lambda i: (0, 0)),
            pl.BlockSpec((h, c), lambda i: (0, 0)),
        ],
        out_specs=(
            pl.BlockSpec((tm, k), lambda i: (i, 0)),
            pl.BlockSpec((tm, c), lambda i: (i, 0)),
        ),
        compiler_params=pltpu.CompilerParams(
            dimension_semantics=("parallel",),
            vmem_limit_bytes=_VMEM_LIMIT_BYTES,
        ),
    )(adj_f32, b, w)


def _fused_layer(adj_b, b, w, *, tm, out_dtype):
    m, k = adj_b.shape
    h = b.shape[1]
    c = w.shape[1]
    return pl.pallas_call(
        _layer_kernel,
        out_shape=jax.ShapeDtypeStruct((m, c), out_dtype),
        grid=(m // tm,),
        in_specs=[
            pl.BlockSpec((tm, k), lambda i: (i, 0)),
            pl.BlockSpec((k, h), lambda i: (0, 0)),
            pl.BlockSpec((h, c), lambda i: (0, 0)),
        ],
        out_specs=pl.BlockSpec((tm, c), lambda i: (i, 0)),
        compiler_params=pltpu.CompilerParams(
            dimension_semantics=("parallel",),
            vmem_limit_bytes=_VMEM_LIMIT_BYTES,
        ),
    )(adj_b, b, w)


def _final_matmul(adj_b, b, *, tm, out_dtype):
    m, k = adj_b.shape
    c = b.shape[1]
    return pl.pallas_call(
        _final_kernel,
        out_shape=jax.ShapeDtypeStruct((m, c), out_dtype),
        grid=(m // tm,),
        in_specs=[
            pl.BlockSpec((tm, k), lambda i: (i, 0)),
            pl.BlockSpec((k, c), lambda i: (0, 0)),
        ],
        out_specs=pl.BlockSpec((tm, c), lambda i: (i, 0)),
        compiler_params=pltpu.CompilerParams(
            dimension_semantics=("parallel",),
            vmem_limit_bytes=_VMEM_LIMIT_BYTES,
        ),
    )(adj_b, b)


def kernel(W1, W2, Wmid0, x, adj):
    del x
    n = adj.shape[0]
    assert n % 256 == 0, adj.shape
    tm = 512

    w1_b = W1.astype(jnp.bfloat16)
    wm_b = Wmid0.astype(jnp.bfloat16)
    w2_b = W2.astype(jnp.bfloat16)

    adj_b, pre1 = _layer1(adj, w1_b, wm_b, tm=tm)
    pre2 = _fused_layer(adj_b, pre1, w2_b, tm=tm, out_dtype=jnp.bfloat16)
    return _final_matmul(adj_b, pre2, tm=tm, out_dtype=jnp.float32)


# speedup vs baseline: 4.3004x; 1.0448x over previous
import functools

import jax
import jax.numpy as jnp
from jax.experimental import pallas as pl
from jax.experimental.pallas import tpu as pltpu

_VMEM_LIMIT_BYTES = 64 * 1024 * 1024


def _layer1_kernel(adj_ref, b_ref, w_ref, adjb_ref, out_ref):
    adj_b = adj_ref[...].astype(jnp.bfloat16)
    adjb_ref[...] = adj_b
    h = jnp.dot(adj_b, b_ref[...], preferred_element_type=jnp.float32)
    h = jnp.maximum(h, 0.0).astype(jnp.bfloat16)
    out_ref[...] = jnp.dot(
        h, w_ref[...], preferred_element_type=jnp.float32
    ).astype(out_ref.dtype)


def _layer_kernel(adj_ref, b_ref, w_ref, out_ref):
    h = jnp.dot(adj_ref[...], b_ref[...], preferred_element_type=jnp.float32)
    h = jnp.maximum(h, 0.0).astype(jnp.bfloat16)
    out_ref[...] = jnp.dot(
        h, w_ref[...], preferred_element_type=jnp.float32
    ).astype(out_ref.dtype)


def _final_kernel(adj_ref, b_ref, out_ref):
    out_ref[...] = jnp.dot(
        adj_ref[...], b_ref[...], preferred_element_type=jnp.float32
    ).astype(out_ref.dtype)


def _layer1(adj_f32, b, w, *, tm):
    m, k = adj_f32.shape
    h = b.shape[1]
    c = w.shape[1]
    return pl.pallas_call(
        _layer1_kernel,
        out_shape=(
            jax.ShapeDtypeStruct((m, k), jnp.bfloat16),
            jax.ShapeDtypeStruct((m, c), jnp.bfloat16),
        ),
        grid=(m // tm,),
        in_specs=[
            pl.BlockSpec((tm, k), lambda i: (i, 0)),
            pl.BlockSpec((k, h), lambda i: (0, 0)),
            pl.BlockSpec((h, c), lambda i: (0, 0)),
        ],
        out_specs=(
            pl.BlockSpec((tm, k), lambda i: (i, 0)),
            pl.BlockSpec((tm, c), lambda i: (i, 0)),
        ),
        compiler_params=pltpu.CompilerParams(
            dimension_semantics=("parallel",),
            vmem_limit_bytes=_VMEM_LIMIT_BYTES,
        ),
    )(adj_f32, b, w)


def _fused_layer(adj_b, b, w, *, tm, out_dtype):
    m, k = adj_b.shape
    h = b.shape[1]
    c = w.shape[1]
    return pl.pallas_call(
        _layer_kernel,
        out_shape=jax.ShapeDtypeStruct((m, c), out_dtype),
        grid=(m // tm,),
        in_specs=[
            pl.BlockSpec((tm, k), lambda i: (i, 0)),
            pl.BlockSpec((k, h), lambda i: (0, 0)),
            pl.BlockSpec((h, c), lambda i: (0, 0)),
        ],
        out_specs=pl.BlockSpec((tm, c), lambda i: (i, 0)),
        compiler_params=pltpu.CompilerParams(
            dimension_semantics=("parallel",),
            vmem_limit_bytes=_VMEM_LIMIT_BYTES,
        ),
    )(adj_b, b, w)


def _final_matmul(adj_b, b, *, tm, out_dtype):
    m, k = adj_b.shape
    c = b.shape[1]
    return pl.pallas_call(
        _final_kernel,
        out_shape=jax.ShapeDtypeStruct((m, c), out_dtype),
        grid=(m // tm,),
        in_specs=[
            pl.BlockSpec((tm, k), lambda i: (i, 0)),
            pl.BlockSpec((k, c), lambda i: (0, 0)),
        ],
        out_specs=pl.BlockSpec((tm, c), lambda i: (i, 0)),
        compiler_params=pltpu.CompilerParams(
            dimension_semantics=("parallel",),
            vmem_limit_bytes=_VMEM_LIMIT_BYTES,
        ),
    )(adj_b, b)


def kernel(W1, W2, Wmid0, x, adj):
    del x
    n = adj.shape[0]
    assert n % 256 == 0, adj.shape
    tm = 1024

    w1_b = W1.astype(jnp.bfloat16)
    wm_b = Wmid0.astype(jnp.bfloat16)
    w2_b = W2.astype(jnp.bfloat16)

    adj_b, pre1 = _layer1(adj, w1_b, wm_b, tm=tm)
    pre2 = _fused_layer(adj_b, pre1, w2_b, tm=tm, out_dtype=jnp.bfloat16)
    return _final_matmul(adj_b, pre2, tm=tm, out_dtype=jnp.float32)


# speedup vs baseline: 5.1590x; 1.1997x over previous
import functools

import jax
import jax.numpy as jnp
from jax.experimental import pallas as pl
from jax.experimental.pallas import tpu as pltpu

_VMEM_LIMIT_BYTES = 64 * 1024 * 1024


def _layer1_kernel(adj_ref, b_ref, w_ref, adjb_ref, out_ref):
    adj_b = adj_ref[...].astype(jnp.bfloat16)
    adjb_ref[...] = adj_b
    h = jnp.dot(adj_b, b_ref[...].astype(jnp.bfloat16),
                preferred_element_type=jnp.float32)
    h = jnp.maximum(h, 0.0).astype(jnp.bfloat16)
    out_ref[...] = jnp.dot(
        h, w_ref[...].astype(jnp.bfloat16), preferred_element_type=jnp.float32
    ).astype(out_ref.dtype)


def _layer_kernel(adj_ref, b_ref, w_ref, out_ref):
    h = jnp.dot(adj_ref[...], b_ref[...], preferred_element_type=jnp.float32)
    h = jnp.maximum(h, 0.0).astype(jnp.bfloat16)
    out_ref[...] = jnp.dot(
        h, w_ref[...].astype(jnp.bfloat16), preferred_element_type=jnp.float32
    ).astype(out_ref.dtype)


def _final_kernel(adj_ref, b_ref, out_ref):
    out_ref[...] = jnp.dot(
        adj_ref[...], b_ref[...], preferred_element_type=jnp.float32
    ).astype(out_ref.dtype)


def _layer1(adj_f32, b, w, *, tm):
    m, k = adj_f32.shape
    h = b.shape[1]
    c = w.shape[1]
    return pl.pallas_call(
        _layer1_kernel,
        out_shape=(
            jax.ShapeDtypeStruct((m, k), jnp.bfloat16),
            jax.ShapeDtypeStruct((m, c), jnp.bfloat16),
        ),
        grid=(m // tm,),
        in_specs=[
            pl.BlockSpec((tm, k), lambda i: (i, 0)),
            pl.BlockSpec((k, h), lambda i: (0, 0)),
            pl.BlockSpec((h, c), lambda i: (0, 0)),
        ],
        out_specs=(
            pl.BlockSpec((tm, k), lambda i: (i, 0)),
            pl.BlockSpec((tm, c), lambda i: (i, 0)),
        ),
        compiler_params=pltpu.CompilerParams(
            dimension_semantics=("parallel",),
            vmem_limit_bytes=_VMEM_LIMIT_BYTES,
        ),
    )(adj_f32, b, w)


def _fused_layer(adj_b, b, w, *, tm, out_dtype):
    m, k = adj_b.shape
    h = b.shape[1]
    c = w.shape[1]
    return pl.pallas_call(
        _layer_kernel,
        out_shape=jax.ShapeDtypeStruct((m, c), out_dtype),
        grid=(m // tm,),
        in_specs=[
            pl.BlockSpec((tm, k), lambda i: (i, 0)),
            pl.BlockSpec((k, h), lambda i: (0, 0)),
            pl.BlockSpec((h, c), lambda i: (0, 0)),
        ],
        out_specs=pl.BlockSpec((tm, c), lambda i: (i, 0)),
        compiler_params=pltpu.CompilerParams(
            dimension_semantics=("parallel",),
            vmem_limit_bytes=_VMEM_LIMIT_BYTES,
        ),
    )(adj_b, b, w)


def _final_matmul(adj_b, b, *, tm, out_dtype):
    m, k = adj_b.shape
    c = b.shape[1]
    return pl.pallas_call(
        _final_kernel,
        out_shape=jax.ShapeDtypeStruct((m, c), out_dtype),
        grid=(m // tm,),
        in_specs=[
            pl.BlockSpec((tm, k), lambda i: (i, 0)),
            pl.BlockSpec((k, c), lambda i: (0, 0)),
        ],
        out_specs=pl.BlockSpec((tm, c), lambda i: (i, 0)),
        compiler_params=pltpu.CompilerParams(
            dimension_semantics=("parallel",),
            vmem_limit_bytes=_VMEM_LIMIT_BYTES,
        ),
    )(adj_b, b)


def kernel(W1, W2, Wmid0, x, adj):
    del x
    n = adj.shape[0]
    assert n % 256 == 0, adj.shape
    tm = 1024

    adj_b, pre1 = _layer1(adj, W1, Wmid0, tm=tm)
    pre2 = _fused_layer(adj_b, pre1, W2, tm=tm, out_dtype=jnp.bfloat16)
    return _final_matmul(adj_b, pre2, tm=tm, out_dtype=jnp.float32)
